# baseline (speedup 1.0000x reference)
import functools

import jax
import jax.numpy as jnp
from jax.experimental import pallas as pl
from jax.experimental.pallas import tpu as pltpu


def _mm_kernel(x_ref, w_ref, out_ref, wbf_ref):
    @pl.when((pl.program_id(0) == 0) & (pl.program_id(1) == 0))
    def _cast_weight_once():
        wbf_ref[...] = w_ref[...].astype(jnp.bfloat16)

    out_ref[0] = jax.lax.dot_general(
        wbf_ref[...], x_ref[0].astype(jnp.bfloat16),
        (((1,), (1,)), ((), ())),
        preferred_element_type=jnp.float32,
    )


@jax.jit
def kernel(x, weight):
    B, S, I = x.shape
    O = weight.shape[0]
    S_BLK = min(S, 512)

    grid = (B, S // S_BLK)
    return pl.pallas_call(
        _mm_kernel,
        grid=grid,
        in_specs=[
            pl.BlockSpec((1, S_BLK, I), lambda b, s: (b, s, 0)),
            pl.BlockSpec((O, I), lambda b, s: (0, 0)),
        ],
        out_specs=pl.BlockSpec((1, O, S_BLK), lambda b, s: (b, 0, s)),
        out_shape=jax.ShapeDtypeStruct((B, O, S), jnp.float32),
        scratch_shapes=[pltpu.VMEM((O, I), jnp.bfloat16)],
        compiler_params=pltpu.CompilerParams(
            dimension_semantics=("parallel", "arbitrary"),
        ),
    )(x, weight)


# speedup vs baseline: 1.0255x; 1.0255x over previous
import functools

import jax
import jax.numpy as jnp
from jax.experimental import pallas as pl
from jax.experimental.pallas import tpu as pltpu


def _mm_kernel(x_ref, w_ref, out_ref):
    xt = x_ref[0].T
    out_ref[0] = jax.lax.dot_general(
        w_ref[...], xt,
        (((1,), (0,)), ((), ())),
        preferred_element_type=jnp.float32,
    )


@jax.jit
def kernel(x, weight):
    B, S, I = x.shape
    O = weight.shape[0]
    S_BLK = min(S, 512)

    grid = (B, S // S_BLK)
    return pl.pallas_call(
        _mm_kernel,
        grid=grid,
        in_specs=[
            pl.BlockSpec((1, S_BLK, I), lambda b, s: (b, s, 0)),
            pl.BlockSpec((O, I), lambda b, s: (0, 0)),
        ],
        out_specs=pl.BlockSpec((1, O, S_BLK), lambda b, s: (b, 0, s)),
        out_shape=jax.ShapeDtypeStruct((B, O, S), jnp.float32),
        compiler_params=pltpu.CompilerParams(
            dimension_semantics=("parallel", "arbitrary"),
        ),
    )(x, weight)


# speedup vs baseline: 1.0434x; 1.0174x over previous
import jax
import jax.numpy as jnp
from jax.experimental import pallas as pl
from jax.experimental.pallas import tpu as pltpu

N_CHUNK = 4


def _mm_kernel(x_ref, w_hbm, out_ref, wv_ref, sems):
    first = (pl.program_id(0) == 0) & (pl.program_id(1) == 0)
    O = wv_ref.shape[0]
    C = O // N_CHUNK

    @pl.when(first)
    def _first_step():
        for q in range(N_CHUNK):
            pltpu.make_async_copy(
                w_hbm.at[pl.ds(q * C, C), :], wv_ref.at[pl.ds(q * C, C), :],
                sems.at[q]).start()
        for q in range(N_CHUNK):
            pltpu.make_async_copy(
                w_hbm.at[pl.ds(q * C, C), :], wv_ref.at[pl.ds(q * C, C), :],
                sems.at[q]).wait()
            out_ref[0, pl.ds(q * C, C), :] = jax.lax.dot_general(
                wv_ref[pl.ds(q * C, C), :], x_ref[0],
                (((1,), (1,)), ((), ())), preferred_element_type=jnp.float32)

    @pl.when(jnp.logical_not(first))
    def _rest():
        out_ref[0] = jax.lax.dot_general(
            wv_ref[...], x_ref[0],
            (((1,), (1,)), ((), ())), preferred_element_type=jnp.float32)


@jax.jit
def kernel(x, weight):
    B, S, I = x.shape
    O = weight.shape[0]
    S_BLK = min(S, 512)

    grid = (B, S // S_BLK)
    return pl.pallas_call(
        _mm_kernel,
        grid=grid,
        in_specs=[
            pl.BlockSpec((1, S_BLK, I), lambda b, s: (b, s, 0)),
            pl.BlockSpec(memory_space=pl.ANY),
        ],
        out_specs=pl.BlockSpec((1, O, S_BLK), lambda b, s: (b, 0, s)),
        out_shape=jax.ShapeDtypeStruct((B, O, S), jnp.float32),
        scratch_shapes=[
            pltpu.VMEM((O, I), jnp.float32),
            pltpu.SemaphoreType.DMA((N_CHUNK,)),
        ],
        compiler_params=pltpu.CompilerParams(
            dimension_semantics=("parallel", "arbitrary"),
        ),
    )(x, weight)
